# baseline (speedup 1.0000x reference)
import functools

import jax
import jax.numpy as jnp
from jax import lax
from jax.experimental import pallas as pl
from jax.experimental.pallas import tpu as pltpu
from jax.experimental.pallas import tpu_sc as plsc

_N_NODES = 10000
_N_EDGES = 320000
_N_PAD = 10240
_LANES = 16
_BLK = 1024


def _partial_hist(edges):
    info = plsc.get_sparse_core_info()
    nc, ns = info.num_cores, info.num_subcores
    nw = nc * ns
    e_per_w = _N_EDGES // nw
    mesh = plsc.VectorSubcoreMesh(core_axis_name="c", subcore_axis_name="s")

    @functools.partial(
        pl.kernel,
        mesh=mesh,
        out_type=jax.ShapeDtypeStruct((nw, _N_PAD), jnp.int32),
        scratch_types=[
            pltpu.VMEM((e_per_w,), jnp.int32),
            pltpu.VMEM((_N_PAD,), jnp.int32),
        ],
        compiler_params=pltpu.CompilerParams(needs_layout_passes=False),
    )
    def hist(edge_hbm, out_hbm, idx_v, cnt_v):
        wid = lax.axis_index("s") * nc + lax.axis_index("c")
        base = wid * e_per_w
        pltpu.sync_copy(edge_hbm.at[pl.ds(base, e_per_w)], idx_v)

        zeros = jnp.zeros((_LANES,), jnp.int32)

        def zero_body(i, carry):
            for u in range(8):
                cnt_v[pl.ds((i * 8 + u) * _LANES, _LANES)] = zeros
            return carry

        lax.fori_loop(0, _N_PAD // (8 * _LANES), zero_body, 0)

        ones = jnp.ones((_LANES,), jnp.int32)

        def add_body(i, carry):
            for u in range(5):
                idx = idx_v[pl.ds((i * 5 + u) * _LANES, _LANES)]
                plsc.addupdate_scatter(cnt_v, [idx], ones)
            return carry

        lax.fori_loop(0, e_per_w // (5 * _LANES), add_body, 0)

        pltpu.sync_copy(cnt_v, out_hbm.at[wid])

    return hist(edges)


def _mlp_body(h_ref, w1_ref, b1_ref, w2_ref, b2_ref, w3_ref, b3_ref, o_ref):
    x = jnp.maximum(
        jnp.dot(h_ref[...], w1_ref[...], preferred_element_type=jnp.float32)
        + b1_ref[...], 0.0)
    x = jnp.maximum(
        jnp.dot(x, w2_ref[...], preferred_element_type=jnp.float32)
        + b2_ref[...], 0.0)
    o_ref[...] = (jnp.dot(x, w3_ref[...], preferred_element_type=jnp.float32)
                  + b3_ref[...])


def _combine_body(p_ref, h_ref, d_ref, o_ref):
    p = p_ref[...].astype(jnp.float32)
    ones = jnp.ones((p.shape[0], 128), jnp.float32)
    cnt = lax.dot_general(p, ones, (((0,), (0,)), ((), ())),
                          preferred_element_type=jnp.float32)
    o_ref[...] = h_ref[...] + cnt * d_ref[...]


def kernel(h, edge_index, W1, b1, W2, b2, W3, b3):
    part = _partial_hist(edge_index.astype(jnp.int32).reshape(-1))

    grid = _N_PAD // _BLK
    delta = pl.pallas_call(
        _mlp_body,
        grid=(grid,),
        in_specs=[
            pl.BlockSpec((_BLK, 128), lambda i: (i, 0)),
            pl.BlockSpec((128, 128), lambda i: (0, 0)),
            pl.BlockSpec((1, 128), lambda i: (0, 0)),
            pl.BlockSpec((128, 256), lambda i: (0, 0)),
            pl.BlockSpec((1, 256), lambda i: (0, 0)),
            pl.BlockSpec((256, 128), lambda i: (0, 0)),
            pl.BlockSpec((1, 128), lambda i: (0, 0)),
        ],
        out_specs=pl.BlockSpec((_BLK, 128), lambda i: (i, 0)),
        out_shape=jax.ShapeDtypeStruct((_N_NODES, 128), jnp.float32),
    )(h, W1, b1.reshape(1, -1), W2, b2.reshape(1, -1), W3, b3.reshape(1, -1))

    nw = part.shape[0]
    out = pl.pallas_call(
        _combine_body,
        grid=(grid,),
        in_specs=[
            pl.BlockSpec((nw, _BLK), lambda i: (0, i)),
            pl.BlockSpec((_BLK, 128), lambda i: (i, 0)),
            pl.BlockSpec((_BLK, 128), lambda i: (i, 0)),
        ],
        out_specs=pl.BlockSpec((_BLK, 128), lambda i: (i, 0)),
        out_shape=jax.ShapeDtypeStruct((_N_NODES, 128), jnp.float32),
    )(part, h, delta)
    return out


# speedup vs baseline: 1.1575x; 1.1575x over previous
import functools

import jax
import jax.numpy as jnp
from jax import lax
from jax.experimental import pallas as pl
from jax.experimental.pallas import tpu as pltpu
from jax.experimental.pallas import tpu_sc as plsc

_N_NODES = 10000
_N_EDGES = 320000
_N_PAD = 10240
_LANES = 16
_BLK = 1024


_CHUNK = 9984
_TAIL = _N_EDGES - 32 * _CHUNK


def _partial_hist(edges):
    info = plsc.get_sparse_core_info()
    nc, ns = info.num_cores, info.num_subcores
    nw = nc * ns
    mesh = plsc.VectorSubcoreMesh(core_axis_name="c", subcore_axis_name="s")

    @functools.partial(
        pl.kernel,
        mesh=mesh,
        out_type=jax.ShapeDtypeStruct((nw, _N_PAD), jnp.int32),
        scratch_types=[
            pltpu.VMEM((2, _CHUNK), jnp.int32),
            pltpu.VMEM((2, _TAIL), jnp.int32),
            pltpu.VMEM((_N_PAD,), jnp.int32),
        ],
        compiler_params=pltpu.CompilerParams(needs_layout_passes=False),
    )
    def hist(edge_hbm, out_hbm, idx_v, tail_v, cnt_v):
        wid = lax.axis_index("s") * nc + lax.axis_index("c")
        pltpu.sync_copy(edge_hbm.at[:, pl.ds(wid * _CHUNK, _CHUNK)], idx_v)

        zeros = jnp.zeros((_LANES,), jnp.int32)

        def zero_body(i, carry):
            for u in range(8):
                cnt_v[pl.ds((i * 8 + u) * _LANES, _LANES)] = zeros
            return carry

        lax.fori_loop(0, _N_PAD // (8 * _LANES), zero_body, 0)

        ones = jnp.ones((_LANES,), jnp.int32)

        def add_body(i, carry):
            for u in range(6):
                idx = idx_v[0, pl.ds((i * 6 + u) * _LANES, _LANES)]
                plsc.addupdate_scatter(cnt_v, [idx], ones)
            return carry

        lax.fori_loop(0, _CHUNK // (6 * _LANES), add_body, 0)

        @pl.when(wid == nw - 1)
        def _tail():
            pltpu.sync_copy(edge_hbm.at[:, pl.ds(32 * _CHUNK, _TAIL)],
                            tail_v)

            def tail_body(i, carry):
                idx = tail_v[0, pl.ds(i * _LANES, _LANES)]
                plsc.addupdate_scatter(cnt_v, [idx], ones)
                return carry

            lax.fori_loop(0, _TAIL // _LANES, tail_body, 0)

        pltpu.sync_copy(cnt_v, out_hbm.at[wid])

    return hist(edges)


def _mlp_body(p_ref, h_ref, w1_ref, b1_ref, w2_ref, b2_ref, w3_ref, b3_ref,
              o_ref):
    hb = h_ref[...]
    x = jnp.maximum(
        jnp.dot(hb, w1_ref[...], preferred_element_type=jnp.float32)
        + b1_ref[...], 0.0)
    x = jnp.maximum(
        jnp.dot(x, w2_ref[...], preferred_element_type=jnp.float32)
        + b2_ref[...], 0.0)
    delta = (jnp.dot(x, w3_ref[...], preferred_element_type=jnp.float32)
             + b3_ref[...])
    p = p_ref[...].astype(jnp.float32)
    ones = jnp.ones((p.shape[0], 128), jnp.float32)
    cnt = lax.dot_general(p, ones, (((0,), (0,)), ((), ())),
                          preferred_element_type=jnp.float32)
    o_ref[...] = hb + cnt * delta


def kernel(h, edge_index, W1, b1, W2, b2, W3, b3):
    part = _partial_hist(edge_index.astype(jnp.int32))

    nw = part.shape[0]
    grid = _N_PAD // _BLK
    out = pl.pallas_call(
        _mlp_body,
        grid=(grid,),
        in_specs=[
            pl.BlockSpec((nw, _BLK), lambda i: (0, i)),
            pl.BlockSpec((_BLK, 128), lambda i: (i, 0)),
            pl.BlockSpec((128, 128), lambda i: (0, 0)),
            pl.BlockSpec((1, 128), lambda i: (0, 0)),
            pl.BlockSpec((128, 256), lambda i: (0, 0)),
            pl.BlockSpec((1, 256), lambda i: (0, 0)),
            pl.BlockSpec((256, 128), lambda i: (0, 0)),
            pl.BlockSpec((1, 128), lambda i: (0, 0)),
        ],
        out_specs=pl.BlockSpec((_BLK, 128), lambda i: (i, 0)),
        out_shape=jax.ShapeDtypeStruct((_N_NODES, 128), jnp.float32),
    )(part, h, W1, b1.reshape(1, -1), W2, b2.reshape(1, -1),
      W3, b3.reshape(1, -1))
    return out


# speedup vs baseline: 1.2327x; 1.0649x over previous
import functools

import jax
import jax.numpy as jnp
from jax import lax
from jax.experimental import pallas as pl
from jax.experimental.pallas import tpu as pltpu
from jax.experimental.pallas import tpu_sc as plsc

_N_NODES = 10000
_N_EDGES = 320000
_N_PAD = 10240
_LANES = 16
_BLK = 2048


_CHUNK = 9984
_TAIL = _N_EDGES - 32 * _CHUNK


def _partial_hist(edges):
    info = plsc.get_sparse_core_info()
    nc, ns = info.num_cores, info.num_subcores
    nw = nc * ns
    mesh = plsc.VectorSubcoreMesh(core_axis_name="c", subcore_axis_name="s")

    @functools.partial(
        pl.kernel,
        mesh=mesh,
        out_type=jax.ShapeDtypeStruct((nw, _N_PAD), jnp.int32),
        scratch_types=[
            pltpu.VMEM((2, _CHUNK), jnp.int32),
            pltpu.VMEM((2, _TAIL), jnp.int32),
            pltpu.VMEM((_N_PAD,), jnp.int32),
        ],
        compiler_params=pltpu.CompilerParams(needs_layout_passes=False),
    )
    def hist(edge_hbm, out_hbm, idx_v, tail_v, cnt_v):
        wid = lax.axis_index("s") * nc + lax.axis_index("c")
        pltpu.sync_copy(edge_hbm.at[:, pl.ds(wid * _CHUNK, _CHUNK)], idx_v)

        zeros = jnp.zeros((_LANES,), jnp.int32)

        def zero_body(i, carry):
            for u in range(8):
                cnt_v[pl.ds((i * 8 + u) * _LANES, _LANES)] = zeros
            return carry

        lax.fori_loop(0, _N_PAD // (8 * _LANES), zero_body, 0)

        ones = jnp.ones((_LANES,), jnp.int32)

        def add_body(i, carry):
            for u in range(6):
                idx = idx_v[0, pl.ds((i * 6 + u) * _LANES, _LANES)]
                plsc.addupdate_scatter(cnt_v, [idx], ones)
            return carry

        lax.fori_loop(0, _CHUNK // (6 * _LANES), add_body, 0)

        @pl.when(wid == nw - 1)
        def _tail():
            pltpu.sync_copy(edge_hbm.at[:, pl.ds(32 * _CHUNK, _TAIL)],
                            tail_v)

            def tail_body(i, carry):
                idx = tail_v[0, pl.ds(i * _LANES, _LANES)]
                plsc.addupdate_scatter(cnt_v, [idx], ones)
                return carry

            lax.fori_loop(0, _TAIL // _LANES, tail_body, 0)

        pltpu.sync_copy(cnt_v, out_hbm.at[wid])

    return hist(edges)


def _mlp_body(p_ref, h_ref, w1_ref, b1_ref, w2_ref, b2_ref, w3_ref, b3_ref,
              o_ref):
    hb = h_ref[...]
    x = jnp.maximum(
        jnp.dot(hb, w1_ref[...], preferred_element_type=jnp.float32)
        + b1_ref[...], 0.0)
    x = jnp.maximum(
        jnp.dot(x, w2_ref[...], preferred_element_type=jnp.float32)
        + b2_ref[...], 0.0)
    delta = (jnp.dot(x, w3_ref[...], preferred_element_type=jnp.float32)
             + b3_ref[...])
    p = p_ref[...].astype(jnp.float32)
    ones = jnp.ones((p.shape[0], 128), jnp.float32)
    cnt = lax.dot_general(p, ones, (((0,), (0,)), ((), ())),
                          preferred_element_type=jnp.float32)
    o_ref[...] = hb + cnt * delta


def kernel(h, edge_index, W1, b1, W2, b2, W3, b3):
    part = _partial_hist(edge_index.astype(jnp.int32))

    nw = part.shape[0]
    grid = _N_PAD // _BLK
    out = pl.pallas_call(
        _mlp_body,
        grid=(grid,),
        in_specs=[
            pl.BlockSpec((nw, _BLK), lambda i: (0, i)),
            pl.BlockSpec((_BLK, 128), lambda i: (i, 0)),
            pl.BlockSpec((128, 128), lambda i: (0, 0)),
            pl.BlockSpec((1, 128), lambda i: (0, 0)),
            pl.BlockSpec((128, 256), lambda i: (0, 0)),
            pl.BlockSpec((1, 256), lambda i: (0, 0)),
            pl.BlockSpec((256, 128), lambda i: (0, 0)),
            pl.BlockSpec((1, 128), lambda i: (0, 0)),
        ],
        out_specs=pl.BlockSpec((_BLK, 128), lambda i: (i, 0)),
        out_shape=jax.ShapeDtypeStruct((_N_NODES, 128), jnp.float32),
    )(part, h, W1, b1.reshape(1, -1), W2, b2.reshape(1, -1),
      W3, b3.reshape(1, -1))
    return out


# speedup vs baseline: 1.2616x; 1.0235x over previous
import functools

import jax
import jax.numpy as jnp
from jax import lax
from jax.experimental import pallas as pl
from jax.experimental.pallas import tpu as pltpu
from jax.experimental.pallas import tpu_sc as plsc

_N_NODES = 10000
_N_EDGES = 320000
_N_PAD = 10240
_LANES = 16
_BLK = 5120


_CHUNK = 9984
_TAIL = _N_EDGES - 32 * _CHUNK


def _partial_hist(edges):
    info = plsc.get_sparse_core_info()
    nc, ns = info.num_cores, info.num_subcores
    nw = nc * ns
    mesh = plsc.VectorSubcoreMesh(core_axis_name="c", subcore_axis_name="s")

    @functools.partial(
        pl.kernel,
        mesh=mesh,
        out_type=jax.ShapeDtypeStruct((nw, _N_PAD), jnp.int32),
        scratch_types=[
            pltpu.VMEM((2, _CHUNK), jnp.int32),
            pltpu.VMEM((2, _TAIL), jnp.int32),
            pltpu.VMEM((_N_PAD,), jnp.int32),
        ],
        compiler_params=pltpu.CompilerParams(needs_layout_passes=False),
    )
    def hist(edge_hbm, out_hbm, idx_v, tail_v, cnt_v):
        wid = lax.axis_index("s") * nc + lax.axis_index("c")
        pltpu.sync_copy(edge_hbm.at[:, pl.ds(wid * _CHUNK, _CHUNK)], idx_v)

        zeros = jnp.zeros((_LANES,), jnp.int32)

        def zero_body(i, carry):
            for u in range(8):
                cnt_v[pl.ds((i * 8 + u) * _LANES, _LANES)] = zeros
            return carry

        lax.fori_loop(0, _N_PAD // (8 * _LANES), zero_body, 0)

        ones = jnp.ones((_LANES,), jnp.int32)

        def add_body(i, carry):
            for u in range(6):
                idx = idx_v[0, pl.ds((i * 6 + u) * _LANES, _LANES)]
                plsc.addupdate_scatter(cnt_v, [idx], ones)
            return carry

        lax.fori_loop(0, _CHUNK // (6 * _LANES), add_body, 0)

        @pl.when(wid == nw - 1)
        def _tail():
            pltpu.sync_copy(edge_hbm.at[:, pl.ds(32 * _CHUNK, _TAIL)],
                            tail_v)

            def tail_body(i, carry):
                idx = tail_v[0, pl.ds(i * _LANES, _LANES)]
                plsc.addupdate_scatter(cnt_v, [idx], ones)
                return carry

            lax.fori_loop(0, _TAIL // _LANES, tail_body, 0)

        pltpu.sync_copy(cnt_v, out_hbm.at[wid])

    return hist(edges)


def _mlp_body(p_ref, h_ref, w1_ref, b1_ref, w2_ref, b2_ref, w3_ref, b3_ref,
              o_ref):
    hb = h_ref[...]
    x = jnp.maximum(
        jnp.dot(hb, w1_ref[...], preferred_element_type=jnp.float32)
        + b1_ref[...], 0.0)
    x = jnp.maximum(
        jnp.dot(x, w2_ref[...], preferred_element_type=jnp.float32)
        + b2_ref[...], 0.0)
    delta = (jnp.dot(x, w3_ref[...], preferred_element_type=jnp.float32)
             + b3_ref[...])
    p = p_ref[...].astype(jnp.float32)
    ones = jnp.ones((p.shape[0], 128), jnp.float32)
    cnt = lax.dot_general(p, ones, (((0,), (0,)), ((), ())),
                          preferred_element_type=jnp.float32)
    o_ref[...] = hb + cnt * delta


def kernel(h, edge_index, W1, b1, W2, b2, W3, b3):
    part = _partial_hist(edge_index.astype(jnp.int32))

    nw = part.shape[0]
    grid = _N_PAD // _BLK
    out = pl.pallas_call(
        _mlp_body,
        grid=(grid,),
        in_specs=[
            pl.BlockSpec((nw, _BLK), lambda i: (0, i)),
            pl.BlockSpec((_BLK, 128), lambda i: (i, 0)),
            pl.BlockSpec((128, 128), lambda i: (0, 0)),
            pl.BlockSpec((1, 128), lambda i: (0, 0)),
            pl.BlockSpec((128, 256), lambda i: (0, 0)),
            pl.BlockSpec((1, 256), lambda i: (0, 0)),
            pl.BlockSpec((256, 128), lambda i: (0, 0)),
            pl.BlockSpec((1, 128), lambda i: (0, 0)),
        ],
        out_specs=pl.BlockSpec((_BLK, 128), lambda i: (i, 0)),
        out_shape=jax.ShapeDtypeStruct((_N_NODES, 128), jnp.float32),
    )(part, h, W1, b1.reshape(1, -1), W2, b2.reshape(1, -1),
      W3, b3.reshape(1, -1))
    return out
